# Initial kernel scaffold; baseline (speedup 1.0000x reference)
import functools

import jax
import jax.numpy as jnp
from jax import lax
from jax.experimental import pallas as pl
from jax.experimental.pallas import tpu as pltpu
from jax.experimental.pallas import tpu_sc as plsc

N = 10000
E = 320000
NC, NS, CH = 2, 16, 128
NPAD = 10240
ROWS_PT = NPAD // NS
EPAD = NC * NS * CH * 79
DW = 8
BLK = 1000


def _make_deg():
    chunks = EPAD // (NC * NS * CH)
    mesh = plsc.VectorSubcoreMesh(core_axis_name="c", subcore_axis_name="s")

    @functools.partial(
        pl.kernel,
        out_type=jax.ShapeDtypeStruct((NC, NPAD, DW), jnp.float32),
        mesh=mesh,
        scratch_types=[
            pltpu.VMEM((CH,), jnp.int32),
            pltpu.VMEM((CH, DW), jnp.float32),
            pltpu.VMEM_SHARED((NPAD, DW), jnp.float32),
        ],
    )
    def deg(dst_hbm, ones_hbm, zeros_hbm, out_hbm, didx, ones_v, acc):
        c = lax.axis_index("c")
        s = lax.axis_index("s")
        r0 = s * ROWS_PT
        pltpu.sync_copy(zeros_hbm, acc.at[pl.ds(r0, ROWS_PT)])
        pltpu.sync_copy(ones_hbm, ones_v)
        plsc.subcore_barrier()
        base = (c * NS + s) * chunks * CH

        def body(j, carry):
            off = base + j * CH
            pltpu.sync_copy(dst_hbm.at[pl.ds(off, CH)], didx)
            pltpu.sync_copy(ones_v, acc.at[didx], add=True)
            return carry

        lax.fori_loop(0, chunks, body, 0)
        plsc.subcore_barrier()
        pltpu.sync_copy(acc.at[pl.ds(r0, ROWS_PT)], out_hbm.at[c, pl.ds(r0, ROWS_PT)])

    return deg


def _make_segsum(dh):
    chunks = EPAD // (NS * CH)
    mesh = plsc.VectorSubcoreMesh(core_axis_name="c", subcore_axis_name="s")

    @functools.partial(
        pl.kernel,
        out_type=jax.ShapeDtypeStruct((NC, NPAD, dh), jnp.float32),
        mesh=mesh,
        scratch_types=[
            pltpu.VMEM((CH,), jnp.int32),
            pltpu.VMEM((CH,), jnp.int32),
            pltpu.VMEM((CH, dh), jnp.float32),
            pltpu.VMEM_SHARED((NPAD, dh), jnp.float32),
            pltpu.SemaphoreType.DMA,
        ],
    )
    def seg(g_hbm, src_hbm, dst_hbm, zeros_hbm, out_hbm, sidx, didx, rows, acc, sem):
        c = lax.axis_index("c")
        s = lax.axis_index("s")
        r0 = s * ROWS_PT
        pltpu.sync_copy(zeros_hbm, acc.at[pl.ds(r0, ROWS_PT)])
        plsc.subcore_barrier()
        base = s * chunks * CH

        def body(j, carry):
            off = base + j * CH
            pltpu.sync_copy(src_hbm.at[pl.ds(off, CH)], sidx)
            pltpu.sync_copy(dst_hbm.at[pl.ds(off, CH)], didx)
            pltpu.async_copy(g_hbm.at[c].at[sidx], rows, sem).wait()
            pltpu.sync_copy(rows, acc.at[didx], add=True)
            return carry

        lax.fori_loop(0, chunks, body, 0)
        plsc.subcore_barrier()
        pltpu.sync_copy(acc.at[pl.ds(r0, ROWS_PT)], out_hbm.at[c, pl.ds(r0, ROWS_PT)])

    return seg


_deg_call = _make_deg()
_seg72 = _make_segsum(72)
_seg128 = _make_segsum(128)


def _prep_body(deg_ref, h_ref, dis_ref, dis2n_ref, g0_ref):
    d = deg_ref[0, :, 0:1] + deg_ref[1, :, 0:1]
    dis = jnp.where(d > 0, 1.0 / jnp.sqrt(jnp.maximum(d, 1e-12)), 0.0)
    dis_ref[...] = dis
    dis2n_ref[...] = -(dis * dis)
    g0 = h_ref[...] * dis
    g0_ref[0] = g0[:, :72]
    g0_ref[1] = g0[:, 72:]


def _prep(deg2, h):
    return pl.pallas_call(
        _prep_body,
        out_shape=(
            jax.ShapeDtypeStruct((N, 1), jnp.float32),
            jax.ShapeDtypeStruct((N, 1), jnp.float32),
            jax.ShapeDtypeStruct((2, N, 72), jnp.float32),
        ),
    )(deg2, h)


def _scale_body(s_ref, t_ref, g_ref):
    g_ref[...] = s_ref[...] * t_ref[...][None, :, :]


def _scale(s, dis2n, dh):
    grid = N // (2 * BLK)
    return pl.pallas_call(
        _scale_body,
        grid=(grid,),
        in_specs=[
            pl.BlockSpec((2, 2 * BLK, dh), lambda i: (0, i, 0)),
            pl.BlockSpec((2 * BLK, 1), lambda i: (i, 0)),
        ],
        out_specs=pl.BlockSpec((2, 2 * BLK, dh), lambda i: (0, i, 0)),
        out_shape=jax.ShapeDtypeStruct((2, N, dh), jnp.float32),
    )(s, dis2n)


def _layer1_body(h_ref, s1_ref, s2_ref, dis_ref, a_ref, b_ref, c_ref, bias_ref,
                 out_ref, gn_ref):
    dis = dis_ref[...]
    t1 = jnp.concatenate([s1_ref[0], s1_ref[1]], axis=1) * dis
    t2 = jnp.concatenate([s2_ref[0], s2_ref[1]], axis=1) * dis
    o = (jnp.dot(h_ref[...], a_ref[...], preferred_element_type=jnp.float32)
         + jnp.dot(t1, b_ref[...], preferred_element_type=jnp.float32)
         + jnp.dot(t2, c_ref[...], preferred_element_type=jnp.float32)
         + bias_ref[...])
    o = jnp.maximum(o, 0.0)
    out_ref[...] = o
    gn = o * dis
    gn_ref[0] = gn[:, :128]
    gn_ref[1] = gn[:, 128:]


def _layer1(h, s1, s2, dis, a, b, c, bias):
    grid = N // BLK
    return pl.pallas_call(
        _layer1_body,
        grid=(grid,),
        in_specs=[
            pl.BlockSpec((BLK, 144), lambda i: (i, 0)),
            pl.BlockSpec((2, BLK, 72), lambda i: (0, i, 0)),
            pl.BlockSpec((2, BLK, 72), lambda i: (0, i, 0)),
            pl.BlockSpec((BLK, 1), lambda i: (i, 0)),
            pl.BlockSpec((144, 256), lambda i: (0, 0)),
            pl.BlockSpec((144, 256), lambda i: (0, 0)),
            pl.BlockSpec((144, 256), lambda i: (0, 0)),
            pl.BlockSpec((1, 256), lambda i: (0, 0)),
        ],
        out_specs=(
            pl.BlockSpec((BLK, 256), lambda i: (i, 0)),
            pl.BlockSpec((2, BLK, 128), lambda i: (0, i, 0)),
        ),
        out_shape=(
            jax.ShapeDtypeStruct((N, 256), jnp.float32),
            jax.ShapeDtypeStruct((2, N, 128), jnp.float32),
        ),
    )(h, s1, s2, dis, a, b, c, bias)


def _layer2_body(h_ref, s1_ref, s2_ref, dis_ref, a_ref, b_ref, c_ref, bias_ref,
                 wmu_ref, bmu_ref, wlv_ref, blv_ref, mu_ref, lv_ref, acc_ref):
    i = pl.program_id(0)
    dis = dis_ref[...]
    t1 = jnp.concatenate([s1_ref[0], s1_ref[1]], axis=1) * dis
    t2 = jnp.concatenate([s2_ref[0], s2_ref[1]], axis=1) * dis
    o = (jnp.dot(h_ref[...], a_ref[...], preferred_element_type=jnp.float32)
         + jnp.dot(t1, b_ref[...], preferred_element_type=jnp.float32)
         + jnp.dot(t2, c_ref[...], preferred_element_type=jnp.float32)
         + bias_ref[...])
    o = jnp.maximum(o, 0.0)
    ps = jnp.sum(o, axis=0, keepdims=True)

    @pl.when(i == 0)
    def _():
        acc_ref[...] = ps

    @pl.when(i > 0)
    def _():
        acc_ref[...] = acc_ref[...] + ps

    @pl.when(i == pl.num_programs(0) - 1)
    def _():
        m = acc_ref[...] * (1.0 / N)
        mu_ref[...] = jnp.dot(m, wmu_ref[...], preferred_element_type=jnp.float32) + bmu_ref[...]
        lv_ref[...] = jnp.dot(m, wlv_ref[...], preferred_element_type=jnp.float32) + blv_ref[...]


def _layer2(h, s1, s2, dis, a, b, c, bias, wmu, bmu, wlv, blv):
    grid = N // BLK
    return pl.pallas_call(
        _layer2_body,
        grid=(grid,),
        in_specs=[
            pl.BlockSpec((BLK, 256), lambda i: (i, 0)),
            pl.BlockSpec((2, BLK, 128), lambda i: (0, i, 0)),
            pl.BlockSpec((2, BLK, 128), lambda i: (0, i, 0)),
            pl.BlockSpec((BLK, 1), lambda i: (i, 0)),
            pl.BlockSpec((256, 256), lambda i: (0, 0)),
            pl.BlockSpec((256, 256), lambda i: (0, 0)),
            pl.BlockSpec((256, 256), lambda i: (0, 0)),
            pl.BlockSpec((1, 256), lambda i: (0, 0)),
            pl.BlockSpec((256, 64), lambda i: (0, 0)),
            pl.BlockSpec((1, 64), lambda i: (0, 0)),
            pl.BlockSpec((256, 64), lambda i: (0, 0)),
            pl.BlockSpec((1, 64), lambda i: (0, 0)),
        ],
        out_specs=(
            pl.BlockSpec((1, 64), lambda i: (0, 0)),
            pl.BlockSpec((1, 64), lambda i: (0, 0)),
        ),
        out_shape=(
            jax.ShapeDtypeStruct((1, 64), jnp.float32),
            jax.ShapeDtypeStruct((1, 64), jnp.float32),
        ),
        scratch_shapes=[pltpu.VMEM((1, 256), jnp.float32)],
    )(h, s1, s2, dis, a, b, c, bias, wmu, bmu, wlv, blv)


def kernel(x, edge_index, lap_pe, W1, b1, W2, b2, Wmu, bmu, Wlv, blv):
    src, dst = edge_index[0], edge_index[1]
    h = jnp.concatenate([x, lap_pe], axis=1)

    pad = EPAD - E
    srcp = jnp.concatenate([src, jnp.zeros((pad,), jnp.int32)])
    dstp = jnp.concatenate([dst, jnp.full((pad,), NPAD - 1, jnp.int32)])

    ones_dw = jnp.ones((CH, DW), jnp.float32)
    zeros_dw = jnp.zeros((ROWS_PT, DW), jnp.float32)
    zeros72 = jnp.zeros((ROWS_PT, 72), jnp.float32)
    zeros128 = jnp.zeros((ROWS_PT, 128), jnp.float32)

    deg_out = _deg_call(dstp, ones_dw, zeros_dw)
    deg2 = deg_out[:, :N, :]
    dis, dis2n, g0 = _prep(deg2, h)

    s1 = _seg72(g0, srcp, dstp, zeros72)[:, :N, :]
    g1 = _scale(s1, dis2n, 72)
    s2 = _seg72(g1, srcp, dstp, zeros72)[:, :N, :]

    a1 = W1[0] - W1[2]
    b1m = -W1[1]
    c1 = -2.0 * W1[2]
    out1, gA = _layer1(h, s1, s2, dis, a1, b1m, c1, b1[None, :])

    s1b = _seg128(gA, srcp, dstp, zeros128)[:, :N, :]
    g1b = _scale(s1b, dis2n, 128)
    s2b = _seg128(g1b, srcp, dstp, zeros128)[:, :N, :]

    a2 = W2[0] - W2[2]
    b2m = -W2[1]
    c2 = -2.0 * W2[2]
    mu, lv = _layer2(out1, s1b, s2b, dis, a2, b2m, c2, b2[None, :],
                     Wmu, bmu[None, :], Wlv, blv[None, :])
    return (mu, lv)


# speedup vs baseline: 4.3115x; 4.3115x over previous
import functools

import jax
import jax.numpy as jnp
from jax import lax
from jax.experimental import pallas as pl
from jax.experimental.pallas import tpu as pltpu
from jax.experimental.pallas import tpu_sc as plsc

N = 10000
E = 320000
NC, NS, CH = 2, 16, 128
NPAD = 10240
ROWS_PT = NPAD // NS
EPAD = NC * NS * CH * 79
DH = 128
BLK = 1000


def _make_deg():
    chunks = EPAD // (NC * NS * CH)
    mesh = plsc.VectorSubcoreMesh(core_axis_name="c", subcore_axis_name="s")

    @functools.partial(
        pl.kernel,
        out_type=jax.ShapeDtypeStruct((NC, NPAD, DH), jnp.float32),
        mesh=mesh,
        scratch_types=[
            pltpu.VMEM((CH,), jnp.int32),
            pltpu.VMEM((CH, DH), jnp.float32),
            pltpu.VMEM_SHARED((NPAD, DH), jnp.float32),
        ],
    )
    def deg(dst_hbm, ones_hbm, zeros_hbm, out_hbm, didx, ones_v, acc):
        c = lax.axis_index("c")
        s = lax.axis_index("s")
        r0 = s * ROWS_PT
        pltpu.sync_copy(zeros_hbm, acc.at[pl.ds(r0, ROWS_PT)])
        pltpu.sync_copy(ones_hbm, ones_v)
        plsc.subcore_barrier()
        base = (c * NS + s) * chunks * CH

        def body(j, carry):
            off = base + j * CH
            pltpu.sync_copy(dst_hbm.at[pl.ds(off, CH)], didx)
            pltpu.sync_copy(ones_v, acc.at[didx], add=True)
            return carry

        lax.fori_loop(0, chunks, body, 0)
        plsc.subcore_barrier()
        pltpu.sync_copy(acc.at[pl.ds(r0, ROWS_PT)], out_hbm.at[c, pl.ds(r0, ROWS_PT)])

    return deg


def _make_segsum():
    chunks = EPAD // (NS * CH)
    mesh = plsc.VectorSubcoreMesh(core_axis_name="c", subcore_axis_name="s")

    @functools.partial(
        pl.kernel,
        out_type=jax.ShapeDtypeStruct((NC, NPAD, DH), jnp.float32),
        mesh=mesh,
        scratch_types=[
            pltpu.VMEM((CH,), jnp.int32),
            pltpu.VMEM((CH,), jnp.int32),
            pltpu.VMEM((CH, DH), jnp.float32),
            pltpu.VMEM_SHARED((NPAD, DH), jnp.float32),
            pltpu.SemaphoreType.DMA,
        ],
    )
    def seg(g_hbm, src_hbm, dst_hbm, zeros_hbm, out_hbm, sidx, didx, rows, acc, sem):
        c = lax.axis_index("c")
        s = lax.axis_index("s")
        r0 = s * ROWS_PT
        pltpu.sync_copy(zeros_hbm, acc.at[pl.ds(r0, ROWS_PT)])
        plsc.subcore_barrier()
        base = s * chunks * CH

        def body(j, carry):
            off = base + j * CH
            pltpu.sync_copy(src_hbm.at[pl.ds(off, CH)], sidx)
            pltpu.sync_copy(dst_hbm.at[pl.ds(off, CH)], didx)
            pltpu.async_copy(g_hbm.at[c].at[sidx], rows, sem).wait()
            pltpu.sync_copy(rows, acc.at[didx], add=True)
            return carry

        lax.fori_loop(0, chunks, body, 0)
        plsc.subcore_barrier()
        pltpu.sync_copy(acc.at[pl.ds(r0, ROWS_PT)], out_hbm.at[c, pl.ds(r0, ROWS_PT)])

    return seg


_deg_call = _make_deg()
_seg = _make_segsum()


def _prep_body(deg_ref, h_ref, dis_ref, dis2n_ref, g0_ref):
    d = deg_ref[0, :, 0:1] + deg_ref[1, :, 0:1]
    dis = jnp.where(d > 0, 1.0 / jnp.sqrt(jnp.maximum(d, 1e-12)), 0.0)
    dis_ref[...] = dis
    dis2n_ref[...] = -(dis * dis)
    g0 = h_ref[...] * dis
    z = jnp.zeros((N, DH - 72), jnp.float32)
    g0_ref[0] = jnp.concatenate([g0[:, :72], z], axis=1)
    g0_ref[1] = jnp.concatenate([g0[:, 72:], z], axis=1)


def _prep(deg2, h):
    return pl.pallas_call(
        _prep_body,
        out_shape=(
            jax.ShapeDtypeStruct((N, 1), jnp.float32),
            jax.ShapeDtypeStruct((N, 1), jnp.float32),
            jax.ShapeDtypeStruct((2, N, DH), jnp.float32),
        ),
    )(deg2, h)


def _scale_body(s_ref, t_ref, g_ref):
    g_ref[...] = s_ref[...] * t_ref[...][None, :, :]


def _scale(s, dis2n):
    grid = N // (2 * BLK)
    return pl.pallas_call(
        _scale_body,
        grid=(grid,),
        in_specs=[
            pl.BlockSpec((2, 2 * BLK, DH), lambda i: (0, i, 0)),
            pl.BlockSpec((2 * BLK, 1), lambda i: (i, 0)),
        ],
        out_specs=pl.BlockSpec((2, 2 * BLK, DH), lambda i: (0, i, 0)),
        out_shape=jax.ShapeDtypeStruct((2, N, DH), jnp.float32),
    )(s, dis2n)


def _layer1_body(h_ref, s1_ref, s2_ref, dis_ref, a_ref, b_ref, c_ref, bias_ref,
                 out_ref, gn_ref):
    dis = dis_ref[...]
    t1 = jnp.concatenate([s1_ref[0, :, :72], s1_ref[1, :, :72]], axis=1) * dis
    t2 = jnp.concatenate([s2_ref[0, :, :72], s2_ref[1, :, :72]], axis=1) * dis
    o = (jnp.dot(h_ref[...], a_ref[...], preferred_element_type=jnp.float32)
         + jnp.dot(t1, b_ref[...], preferred_element_type=jnp.float32)
         + jnp.dot(t2, c_ref[...], preferred_element_type=jnp.float32)
         + bias_ref[...])
    o = jnp.maximum(o, 0.0)
    out_ref[...] = o
    gn = o * dis
    gn_ref[0] = gn[:, :128]
    gn_ref[1] = gn[:, 128:]


def _layer1(h, s1, s2, dis, a, b, c, bias):
    grid = N // BLK
    return pl.pallas_call(
        _layer1_body,
        grid=(grid,),
        in_specs=[
            pl.BlockSpec((BLK, 144), lambda i: (i, 0)),
            pl.BlockSpec((2, BLK, DH), lambda i: (0, i, 0)),
            pl.BlockSpec((2, BLK, DH), lambda i: (0, i, 0)),
            pl.BlockSpec((BLK, 1), lambda i: (i, 0)),
            pl.BlockSpec((144, 256), lambda i: (0, 0)),
            pl.BlockSpec((144, 256), lambda i: (0, 0)),
            pl.BlockSpec((144, 256), lambda i: (0, 0)),
            pl.BlockSpec((1, 256), lambda i: (0, 0)),
        ],
        out_specs=(
            pl.BlockSpec((BLK, 256), lambda i: (i, 0)),
            pl.BlockSpec((2, BLK, 128), lambda i: (0, i, 0)),
        ),
        out_shape=(
            jax.ShapeDtypeStruct((N, 256), jnp.float32),
            jax.ShapeDtypeStruct((2, N, 128), jnp.float32),
        ),
    )(h, s1, s2, dis, a, b, c, bias)


def _layer2_body(h_ref, s1_ref, s2_ref, dis_ref, a_ref, b_ref, c_ref, bias_ref,
                 wmu_ref, bmu_ref, wlv_ref, blv_ref, mu_ref, lv_ref, acc_ref):
    i = pl.program_id(0)
    dis = dis_ref[...]
    t1 = jnp.concatenate([s1_ref[0], s1_ref[1]], axis=1) * dis
    t2 = jnp.concatenate([s2_ref[0], s2_ref[1]], axis=1) * dis
    o = (jnp.dot(h_ref[...], a_ref[...], preferred_element_type=jnp.float32)
         + jnp.dot(t1, b_ref[...], preferred_element_type=jnp.float32)
         + jnp.dot(t2, c_ref[...], preferred_element_type=jnp.float32)
         + bias_ref[...])
    o = jnp.maximum(o, 0.0)
    ps = jnp.sum(o, axis=0, keepdims=True)

    @pl.when(i == 0)
    def _():
        acc_ref[...] = ps

    @pl.when(i > 0)
    def _():
        acc_ref[...] = acc_ref[...] + ps

    @pl.when(i == pl.num_programs(0) - 1)
    def _():
        m = acc_ref[...] * (1.0 / N)
        mu_ref[...] = jnp.dot(m, wmu_ref[...], preferred_element_type=jnp.float32) + bmu_ref[...]
        lv_ref[...] = jnp.dot(m, wlv_ref[...], preferred_element_type=jnp.float32) + blv_ref[...]


def _layer2(h, s1, s2, dis, a, b, c, bias, wmu, bmu, wlv, blv):
    grid = N // BLK
    return pl.pallas_call(
        _layer2_body,
        grid=(grid,),
        in_specs=[
            pl.BlockSpec((BLK, 256), lambda i: (i, 0)),
            pl.BlockSpec((2, BLK, 128), lambda i: (0, i, 0)),
            pl.BlockSpec((2, BLK, 128), lambda i: (0, i, 0)),
            pl.BlockSpec((BLK, 1), lambda i: (i, 0)),
            pl.BlockSpec((256, 256), lambda i: (0, 0)),
            pl.BlockSpec((256, 256), lambda i: (0, 0)),
            pl.BlockSpec((256, 256), lambda i: (0, 0)),
            pl.BlockSpec((1, 256), lambda i: (0, 0)),
            pl.BlockSpec((256, 64), lambda i: (0, 0)),
            pl.BlockSpec((1, 64), lambda i: (0, 0)),
            pl.BlockSpec((256, 64), lambda i: (0, 0)),
            pl.BlockSpec((1, 64), lambda i: (0, 0)),
        ],
        out_specs=(
            pl.BlockSpec((1, 64), lambda i: (0, 0)),
            pl.BlockSpec((1, 64), lambda i: (0, 0)),
        ),
        out_shape=(
            jax.ShapeDtypeStruct((1, 64), jnp.float32),
            jax.ShapeDtypeStruct((1, 64), jnp.float32),
        ),
        scratch_shapes=[pltpu.VMEM((1, 256), jnp.float32)],
    )(h, s1, s2, dis, a, b, c, bias, wmu, bmu, wlv, blv)


def kernel(x, edge_index, lap_pe, W1, b1, W2, b2, Wmu, bmu, Wlv, blv):
    src, dst = edge_index[0], edge_index[1]
    h = jnp.concatenate([x, lap_pe], axis=1)

    pad = EPAD - E
    srcp = jnp.concatenate([src, jnp.zeros((pad,), jnp.int32)])
    dstp = jnp.concatenate([dst, jnp.full((pad,), NPAD - 1, jnp.int32)])

    ones_dh = jnp.ones((CH, DH), jnp.float32)
    zeros_dh = jnp.zeros((ROWS_PT, DH), jnp.float32)

    deg_out = _deg_call(dstp, ones_dh, zeros_dh)
    deg2 = deg_out[:, :N, :]
    dis, dis2n, g0 = _prep(deg2, h)

    s1 = _seg(g0, srcp, dstp, zeros_dh)[:, :N, :]
    g1 = _scale(s1, dis2n)
    s2 = _seg(g1, srcp, dstp, zeros_dh)[:, :N, :]

    a1 = W1[0] - W1[2]
    b1m = -W1[1]
    c1 = -2.0 * W1[2]
    out1, gA = _layer1(h, s1, s2, dis, a1, b1m, c1, b1[None, :])

    s1b = _seg(gA, srcp, dstp, zeros_dh)[:, :N, :]
    g1b = _scale(s1b, dis2n)
    s2b = _seg(g1b, srcp, dstp, zeros_dh)[:, :N, :]

    a2 = W2[0] - W2[2]
    b2m = -W2[1]
    c2 = -2.0 * W2[2]
    mu, lv = _layer2(out1, s1b, s2b, dis, a2, b2m, c2, b2[None, :],
                     Wmu, bmu[None, :], Wlv, blv[None, :])
    return (mu, lv)


# speedup vs baseline: 5.1961x; 1.2052x over previous
import functools

import jax
import jax.numpy as jnp
from jax import lax
from jax.experimental import pallas as pl
from jax.experimental.pallas import tpu as pltpu
from jax.experimental.pallas import tpu_sc as plsc

N = 10000
E = 320000
NC, NS, CH = 2, 16, 128
NPAD = 10240
ROWS_PT = NPAD // NS
GS = 8
NGRP = 20
EPAD = NS * CH * GS * NGRP
DH = 128
BLK = 1000


def _make_deg():
    chunks = EPAD // (NC * NS * CH)
    mesh = plsc.VectorSubcoreMesh(core_axis_name="c", subcore_axis_name="s")

    @functools.partial(
        pl.kernel,
        out_type=jax.ShapeDtypeStruct((NC, NPAD, DH), jnp.float32),
        mesh=mesh,
        scratch_types=[
            pltpu.VMEM((chunks, CH), jnp.int32),
            pltpu.VMEM((CH, DH), jnp.float32),
            pltpu.VMEM_SHARED((NPAD, DH), jnp.float32),
        ],
    )
    def deg(dst_hbm, ones_hbm, zeros_hbm, out_hbm, didx2, ones_v, acc):
        c = lax.axis_index("c")
        s = lax.axis_index("s")
        r0 = s * ROWS_PT
        pltpu.sync_copy(zeros_hbm, acc.at[pl.ds(r0, ROWS_PT)])
        pltpu.sync_copy(ones_hbm, ones_v)
        pltpu.sync_copy(dst_hbm.at[c * NS + s], didx2)
        plsc.subcore_barrier()

        def body(j, carry):
            pltpu.sync_copy(ones_v, acc.at[didx2.at[j]], add=True)
            return carry

        lax.fori_loop(0, chunks, body, 0)
        plsc.subcore_barrier()
        pltpu.sync_copy(acc.at[pl.ds(r0, ROWS_PT)], out_hbm.at[c, pl.ds(r0, ROWS_PT)])

    return deg


def _make_segsum():
    mesh = plsc.VectorSubcoreMesh(core_axis_name="c", subcore_axis_name="s")

    @functools.partial(
        pl.kernel,
        out_type=jax.ShapeDtypeStruct((NC, NPAD, DH), jnp.float32),
        mesh=mesh,
        scratch_types=[
            pltpu.VMEM((GS, CH), jnp.int32),
            pltpu.VMEM((GS, CH), jnp.int32),
            pltpu.VMEM((GS, CH), jnp.int32),
            pltpu.VMEM((GS, CH), jnp.int32),
            pltpu.VMEM((CH, DH), jnp.float32),
            pltpu.VMEM((CH, DH), jnp.float32),
            pltpu.VMEM_SHARED((NPAD, DH), jnp.float32),
            pltpu.SemaphoreType.DMA,
            pltpu.SemaphoreType.DMA,
            pltpu.SemaphoreType.DMA,
        ],
    )
    def seg(g_hbm, src_hbm, dst_hbm, zeros_hbm, out_hbm, sgA, dgA, sgB, dgB,
            rows0, rows1, acc, sem0, sem1, semi):
        c = lax.axis_index("c")
        s = lax.axis_index("s")
        r0 = s * ROWS_PT
        pltpu.sync_copy(zeros_hbm, acc.at[pl.ds(r0, ROWS_PT)])
        plsc.subcore_barrier()

        def process_group(sg, dg):
            rb = (rows0, rows1)
            sb = (sem0, sem1)
            pltpu.async_copy(g_hbm.at[c].at[sg.at[0]], rows0, sem0)
            for k in range(GS):
                if k + 1 < GS:
                    pltpu.async_copy(g_hbm.at[c].at[sg.at[k + 1]],
                                     rb[(k + 1) % 2], sb[(k + 1) % 2])
                pltpu.make_async_copy(g_hbm.at[c].at[sg.at[k]],
                                      rb[k % 2], sb[k % 2]).wait()
                pltpu.sync_copy(rb[k % 2], acc.at[dg.at[k]], add=True)

        pltpu.async_copy(src_hbm.at[s, 0], sgA, semi)
        pltpu.async_copy(dst_hbm.at[s, 0], dgA, semi)

        def body(i, carry):
            g = 2 * i
            pltpu.make_async_copy(src_hbm.at[s, g], sgA, semi).wait()
            pltpu.make_async_copy(dst_hbm.at[s, g], dgA, semi).wait()
            pltpu.async_copy(src_hbm.at[s, g + 1], sgB, semi)
            pltpu.async_copy(dst_hbm.at[s, g + 1], dgB, semi)
            process_group(sgA, dgA)
            pltpu.make_async_copy(src_hbm.at[s, g + 1], sgB, semi).wait()
            pltpu.make_async_copy(dst_hbm.at[s, g + 1], dgB, semi).wait()

            @pl.when(g + 2 < NGRP)
            def _():
                pltpu.async_copy(src_hbm.at[s, g + 2], sgA, semi)
                pltpu.async_copy(dst_hbm.at[s, g + 2], dgA, semi)

            process_group(sgB, dgB)
            return carry

        lax.fori_loop(0, NGRP // 2, body, 0)
        plsc.subcore_barrier()
        pltpu.sync_copy(acc.at[pl.ds(r0, ROWS_PT)], out_hbm.at[c, pl.ds(r0, ROWS_PT)])

    return seg


_deg_call = _make_deg()
_seg = _make_segsum()


def _prep_body(deg_ref, h_ref, dis_ref, dis2n_ref, g0_ref):
    d = deg_ref[0, :, 0:1] + deg_ref[1, :, 0:1]
    dis = jnp.where(d > 0, 1.0 / jnp.sqrt(jnp.maximum(d, 1e-12)), 0.0)
    dis_ref[...] = dis
    dis2n_ref[...] = -(dis * dis)
    g0 = h_ref[...] * dis
    z = jnp.zeros((N, DH - 72), jnp.float32)
    g0_ref[0] = jnp.concatenate([g0[:, :72], z], axis=1)
    g0_ref[1] = jnp.concatenate([g0[:, 72:], z], axis=1)


def _prep(deg2, h):
    return pl.pallas_call(
        _prep_body,
        grid=(1,),
        in_specs=[
            pl.BlockSpec((2, N, DH), lambda i: (0, 0, 0)),
            pl.BlockSpec((N, 144), lambda i: (0, 0)),
        ],
        out_specs=(
            pl.BlockSpec((N, 1), lambda i: (0, 0)),
            pl.BlockSpec((N, 1), lambda i: (0, 0)),
            pl.BlockSpec((2, N, DH), lambda i: (0, 0, 0)),
        ),
        out_shape=(
            jax.ShapeDtypeStruct((N, 1), jnp.float32),
            jax.ShapeDtypeStruct((N, 1), jnp.float32),
            jax.ShapeDtypeStruct((2, N, DH), jnp.float32),
        ),
    )(deg2, h)


def _scale_body(s_ref, t_ref, g_ref):
    g_ref[...] = s_ref[...] * t_ref[...][None, :, :]


def _scale(s, dis2n):
    grid = N // (2 * BLK)
    return pl.pallas_call(
        _scale_body,
        grid=(grid,),
        in_specs=[
            pl.BlockSpec((2, 2 * BLK, DH), lambda i: (0, i, 0)),
            pl.BlockSpec((2 * BLK, 1), lambda i: (i, 0)),
        ],
        out_specs=pl.BlockSpec((2, 2 * BLK, DH), lambda i: (0, i, 0)),
        out_shape=jax.ShapeDtypeStruct((2, N, DH), jnp.float32),
    )(s, dis2n)


def _layer1_body(h_ref, s1_ref, s2_ref, dis_ref, a_ref, b_ref, c_ref, bias_ref,
                 out_ref, gn_ref):
    dis = dis_ref[...]
    t1 = jnp.concatenate([s1_ref[0, :, :72], s1_ref[1, :, :72]], axis=1) * dis
    t2 = jnp.concatenate([s2_ref[0, :, :72], s2_ref[1, :, :72]], axis=1) * dis
    o = (jnp.dot(h_ref[...], a_ref[...], preferred_element_type=jnp.float32)
         + jnp.dot(t1, b_ref[...], preferred_element_type=jnp.float32)
         + jnp.dot(t2, c_ref[...], preferred_element_type=jnp.float32)
         + bias_ref[...])
    o = jnp.maximum(o, 0.0)
    out_ref[...] = o
    gn = o * dis
    gn_ref[0] = gn[:, :128]
    gn_ref[1] = gn[:, 128:]


def _layer1(h, s1, s2, dis, a, b, c, bias):
    grid = N // BLK
    return pl.pallas_call(
        _layer1_body,
        grid=(grid,),
        in_specs=[
            pl.BlockSpec((BLK, 144), lambda i: (i, 0)),
            pl.BlockSpec((2, BLK, DH), lambda i: (0, i, 0)),
            pl.BlockSpec((2, BLK, DH), lambda i: (0, i, 0)),
            pl.BlockSpec((BLK, 1), lambda i: (i, 0)),
            pl.BlockSpec((144, 256), lambda i: (0, 0)),
            pl.BlockSpec((144, 256), lambda i: (0, 0)),
            pl.BlockSpec((144, 256), lambda i: (0, 0)),
            pl.BlockSpec((1, 256), lambda i: (0, 0)),
        ],
        out_specs=(
            pl.BlockSpec((BLK, 256), lambda i: (i, 0)),
            pl.BlockSpec((2, BLK, 128), lambda i: (0, i, 0)),
        ),
        out_shape=(
            jax.ShapeDtypeStruct((N, 256), jnp.float32),
            jax.ShapeDtypeStruct((2, N, 128), jnp.float32),
        ),
    )(h, s1, s2, dis, a, b, c, bias)


def _layer2_body(h_ref, s1_ref, s2_ref, dis_ref, a_ref, b_ref, c_ref, bias_ref,
                 wmu_ref, bmu_ref, wlv_ref, blv_ref, mu_ref, lv_ref, acc_ref):
    i = pl.program_id(0)
    dis = dis_ref[...]
    t1 = jnp.concatenate([s1_ref[0], s1_ref[1]], axis=1) * dis
    t2 = jnp.concatenate([s2_ref[0], s2_ref[1]], axis=1) * dis
    o = (jnp.dot(h_ref[...], a_ref[...], preferred_element_type=jnp.float32)
         + jnp.dot(t1, b_ref[...], preferred_element_type=jnp.float32)
         + jnp.dot(t2, c_ref[...], preferred_element_type=jnp.float32)
         + bias_ref[...])
    o = jnp.maximum(o, 0.0)
    ps = jnp.sum(o, axis=0, keepdims=True)

    @pl.when(i == 0)
    def _():
        acc_ref[...] = ps

    @pl.when(i > 0)
    def _():
        acc_ref[...] = acc_ref[...] + ps

    @pl.when(i == pl.num_programs(0) - 1)
    def _():
        m = acc_ref[...] * (1.0 / N)
        mu_ref[...] = jnp.dot(m, wmu_ref[...], preferred_element_type=jnp.float32) + bmu_ref[...]
        lv_ref[...] = jnp.dot(m, wlv_ref[...], preferred_element_type=jnp.float32) + blv_ref[...]


def _layer2(h, s1, s2, dis, a, b, c, bias, wmu, bmu, wlv, blv):
    grid = N // BLK
    return pl.pallas_call(
        _layer2_body,
        grid=(grid,),
        in_specs=[
            pl.BlockSpec((BLK, 256), lambda i: (i, 0)),
            pl.BlockSpec((2, BLK, 128), lambda i: (0, i, 0)),
            pl.BlockSpec((2, BLK, 128), lambda i: (0, i, 0)),
            pl.BlockSpec((BLK, 1), lambda i: (i, 0)),
            pl.BlockSpec((256, 256), lambda i: (0, 0)),
            pl.BlockSpec((256, 256), lambda i: (0, 0)),
            pl.BlockSpec((256, 256), lambda i: (0, 0)),
            pl.BlockSpec((1, 256), lambda i: (0, 0)),
            pl.BlockSpec((256, 64), lambda i: (0, 0)),
            pl.BlockSpec((1, 64), lambda i: (0, 0)),
            pl.BlockSpec((256, 64), lambda i: (0, 0)),
            pl.BlockSpec((1, 64), lambda i: (0, 0)),
        ],
        out_specs=(
            pl.BlockSpec((1, 64), lambda i: (0, 0)),
            pl.BlockSpec((1, 64), lambda i: (0, 0)),
        ),
        out_shape=(
            jax.ShapeDtypeStruct((1, 64), jnp.float32),
            jax.ShapeDtypeStruct((1, 64), jnp.float32),
        ),
        scratch_shapes=[pltpu.VMEM((1, 256), jnp.float32)],
    )(h, s1, s2, dis, a, b, c, bias, wmu, bmu, wlv, blv)


def kernel(x, edge_index, lap_pe, W1, b1, W2, b2, Wmu, bmu, Wlv, blv):
    src, dst = edge_index[0], edge_index[1]
    h = jnp.concatenate([x, lap_pe], axis=1)

    pad = EPAD - E
    srcp = jnp.concatenate([src, jnp.zeros((pad,), jnp.int32)])
    dstp = jnp.concatenate([dst, jnp.full((pad,), NPAD - 1, jnp.int32)])

    ones_dh = jnp.ones((CH, DH), jnp.float32)
    zeros_dh = jnp.zeros((ROWS_PT, DH), jnp.float32)

    srcp3 = srcp.reshape(NS, NGRP, GS, CH)
    dstp3 = dstp.reshape(NS, NGRP, GS, CH)
    dstp3d = dstp.reshape(NC * NS, EPAD // (NC * NS * CH), CH)

    deg_out = _deg_call(dstp3d, ones_dh, zeros_dh)
    dis, dis2n, g0 = _prep(deg_out, h)

    s1 = _seg(g0, srcp3, dstp3, zeros_dh)
    g1 = _scale(s1, dis2n)
    s2 = _seg(g1, srcp3, dstp3, zeros_dh)

    a1 = W1[0] - W1[2]
    b1m = -W1[1]
    c1 = -2.0 * W1[2]
    out1, gA = _layer1(h, s1, s2, dis, a1, b1m, c1, b1[None, :])

    s1b = _seg(gA, srcp3, dstp3, zeros_dh)
    g1b = _scale(s1b, dis2n)
    s2b = _seg(g1b, srcp3, dstp3, zeros_dh)

    a2 = W2[0] - W2[2]
    b2m = -W2[1]
    c2 = -2.0 * W2[2]
    mu, lv = _layer2(out1, s1b, s2b, dis, a2, b2m, c2, b2[None, :],
                     Wmu, bmu[None, :], Wlv, blv[None, :])
    return (mu, lv)
